# baseline (speedup 1.0000x reference)
import functools

import jax
import jax.numpy as jnp
from jax.experimental import pallas as pl
from jax.experimental.pallas import tpu as pltpu

_D = 768
_F = 3072
_E = 8
_K = 2
_BR = 256


def _router(xf, Wr):
    logits = xf @ Wr
    gate_v, idx = jax.lax.top_k(logits, _K)
    gates = jax.nn.softmax(gate_v, axis=-1)
    return idx, gates


def _compact(idx, gates, N):
    ar = jnp.arange(N, dtype=jnp.int32)
    mask = (idx[:, :, None] == jnp.arange(_E)[None, None, :]).any(axis=1)
    nsel = jnp.sum(mask, axis=0).astype(jnp.int32)
    order = jnp.argsort(~mask, axis=0, stable=True).astype(jnp.int32)
    valid = mask & (ar[:, None] < nsel[None, :])
    pcum = jnp.cumsum(valid.astype(jnp.int32), axis=0) - 1
    counts = jnp.sum(valid, axis=0).astype(jnp.int32)
    g_ne = jnp.einsum("nk,nke->ne", gates,
                      (idx[:, :, None] == jnp.arange(_E)[None, None, :]).astype(gates.dtype))
    src = jnp.zeros((_E, N), jnp.int32)
    dest = jnp.zeros((_E, N), jnp.int32)
    wgt = jnp.zeros((_E, N), jnp.float32)
    for e in range(_E):
        slot = jnp.where(valid[:, e], pcum[:, e], N)
        src = src.at[e, slot].set(order[:, e], mode="drop")
        dest = dest.at[e, slot].set(ar, mode="drop")
        wgt = wgt.at[e, slot].set(g_ne[:, e], mode="drop")
    return counts, src, dest, wgt


def _ffn_body(counts_ref, xg_ref, w1_ref, b1_ref, w2_ref, b2_ref, g_ref,
              be_ref, wgt_ref, yg_ref):
    b = pl.program_id(1)
    e = pl.program_id(0)
    nb = (counts_ref[e] + _BR - 1) // _BR

    @pl.when(b < nb)
    def _():
        xb = xg_ref[...]
        h = jnp.dot(xb, w1_ref[0], preferred_element_type=jnp.float32)
        h = jnp.maximum(h + b1_ref[0, 0][None, :], 0.0)
        o = jnp.dot(h, w2_ref[0], preferred_element_type=jnp.float32)
        o = o + b2_ref[0, 0][None, :]
        hh = xb + o
        mu = jnp.mean(hh, axis=-1, keepdims=True)
        var = jnp.mean((hh - mu) ** 2, axis=-1, keepdims=True)
        y = (hh - mu) * jax.lax.rsqrt(var + 1e-6)
        y = y * g_ref[0, 0][None, :] + be_ref[0, 0][None, :]
        yg_ref[...] = y * wgt_ref[0, 0][:, None]


def _ffn(counts, xg, W1, b1, W2, b2, gamma, beta, wgt):
    N = wgt.shape[1]
    NB = N // _BR
    grid = (_E, NB)

    def blk(e, b, counts_ref):
        nb = (counts_ref[e] + _BR - 1) // _BR
        return jnp.minimum(b, jnp.maximum(nb - 1, 0))

    grid_spec = pltpu.PrefetchScalarGridSpec(
        num_scalar_prefetch=1,
        grid=grid,
        in_specs=[
            pl.BlockSpec((_BR, _D), lambda e, b, c: (e * NB + blk(e, b, c), 0)),
            pl.BlockSpec((1, _D, _F), lambda e, b, c: (e, 0, 0)),
            pl.BlockSpec((1, 1, _F), lambda e, b, c: (e, 0, 0)),
            pl.BlockSpec((1, _F, _D), lambda e, b, c: (e, 0, 0)),
            pl.BlockSpec((1, 1, _D), lambda e, b, c: (e, 0, 0)),
            pl.BlockSpec((1, 1, _D), lambda e, b, c: (e, 0, 0)),
            pl.BlockSpec((1, 1, _D), lambda e, b, c: (e, 0, 0)),
            pl.BlockSpec((1, 1, _BR), lambda e, b, c: (e, 0, blk(e, b, c))),
        ],
        out_specs=pl.BlockSpec((_BR, _D), lambda e, b, c: (e * NB + blk(e, b, c), 0)),
    )
    NB_total = _E * NB
    return pl.pallas_call(
        _ffn_body,
        grid_spec=grid_spec,
        out_shape=jax.ShapeDtypeStruct((NB_total * _BR, _D), jnp.float32),
    )(counts, xg,
      W1, b1.reshape(_E, 1, _F), W2, b2.reshape(_E, 1, _D),
      gamma.reshape(_E, 1, _D), beta.reshape(_E, 1, _D),
      wgt.reshape(_E, 1, N))


def kernel(x, Wr, W1, b1, W2, b2, gamma, beta):
    B, S, D = x.shape
    N = B * S
    xf = x.reshape(N, D)
    idx, gates = _router(xf, Wr)
    counts, src, dest, wgt = _compact(idx, gates, N)
    xg = xf[src.reshape(-1)]
    yg = _ffn(counts, xg, W1, b1, W2, b2, gamma, beta, wgt)
    valid = (jnp.arange(N, dtype=jnp.int32)[None, :] < counts[:, None]).reshape(-1)
    out = jnp.zeros((N, D), jnp.float32)
    out = out.at[dest.reshape(-1)].add(jnp.where(valid[:, None], yg, 0.0))
    return out.reshape(B, S, D)


# speedup vs baseline: 2.1130x; 2.1130x over previous
import functools

import jax
import jax.numpy as jnp
from jax import lax
from jax.experimental import pallas as pl
from jax.experimental.pallas import tpu as pltpu
from jax.experimental.pallas import tpu_sc as plsc

_N = 2048
_D = 768
_F = 3072
_E = 8
_BR = 256
_NB = _N // _BR
_CH = 64
_L = 16
_ZROW = _E * _N


def _router_body(x_ref, wr_ref, i0_ref, i1_ref, g0_ref, g1_ref):
    l = jnp.dot(x_ref[...], wr_ref[...], preferred_element_type=jnp.float32)
    io = lax.broadcasted_iota(jnp.int32, (_N, _E), 1)
    m1 = jnp.max(l, axis=1, keepdims=True)
    a1 = jnp.min(jnp.where(l == m1, io, _E), axis=1, keepdims=True)
    l2 = jnp.where(io == a1, -jnp.inf, l)
    m2 = jnp.max(l2, axis=1, keepdims=True)
    a2 = jnp.min(jnp.where(l2 == m2, io, _E), axis=1, keepdims=True)
    e2 = jnp.exp(m2 - m1)
    den = 1.0 + e2
    i0_ref[...] = a1[:, 0]
    i1_ref[...] = a2[:, 0]
    g0_ref[...] = (1.0 / den)[:, 0]
    g1_ref[...] = (e2 / den)[:, 0]


def _router(xf, Wr):
    return pl.pallas_call(
        _router_body,
        out_shape=(
            jax.ShapeDtypeStruct((_N,), jnp.int32),
            jax.ShapeDtypeStruct((_N,), jnp.int32),
            jax.ShapeDtypeStruct((_N,), jnp.float32),
            jax.ShapeDtypeStruct((_N,), jnp.float32),
        ),
    )(xf, Wr)


def _sc_compact_body(idx0_h, idx1_h, g0_h, g1_h, xf_h,
                     counts_h, wgt_h, pose_h, xg_h,
                     idx0_v, idx1_v, g0_v, g1_v,
                     S_v, srcp_v, wgtp_v, pose_v,
                     cnt_v, idxc_v, rows_v, sem):
    c = lax.axis_index("c")
    s = lax.axis_index("s")

    @pl.when((c == 0) & (s < _E))
    def _():
        e = s
        pltpu.sync_copy(idx0_h, idx0_v)
        pltpu.sync_copy(idx1_h, idx1_v)
        pltpu.sync_copy(g0_h, g0_v)
        pltpu.sync_copy(g1_h, g1_v)
        iota = lax.broadcasted_iota(jnp.int32, (_L,), 0)
        zero_f = jnp.zeros((_L,), jnp.float32)
        neg1 = jnp.full((_L,), -1, jnp.int32)

        def init(j, _):
            wgtp_v[pl.ds(j * _L, _L)] = zero_f
            pose_v[pl.ds(j * _L, _L)] = neg1
            return 0

        lax.fori_loop(0, _N // _L, init, 0)

        def pass1(j, ns):
            tok = j * _L + iota
            i0 = idx0_v[pl.ds(j * _L, _L)]
            i1 = idx1_v[pl.ds(j * _L, _L)]
            sel = (i0 == e) | (i1 == e)
            seli = sel.astype(jnp.int32)
            ranks = ns + plsc.cumsum(seli) - 1
            plsc.store_scatter(S_v, [ranks], tok, mask=sel)
            return ns + plsc.all_reduce_population_count(sel)[0]

        ns = lax.fori_loop(0, _N // _L, pass1, jnp.int32(0))

        def pass2(j, p):
            tok = j * _L + iota
            i0 = idx0_v[pl.ds(j * _L, _L)]
            i1 = idx1_v[pl.ds(j * _L, _L)]
            m0 = i0 == e
            sel = m0 | (i1 == e)
            valid = sel & (tok < ns)
            vi = valid.astype(jnp.int32)
            slots = p + plsc.cumsum(vi) - 1
            srcv = plsc.load_gather(S_v, [tok])
            g = jnp.where(m0, g0_v[pl.ds(j * _L, _L)], g1_v[pl.ds(j * _L, _L)])
            plsc.store_scatter(srcp_v, [slots], srcv, mask=valid)
            plsc.store_scatter(wgtp_v, [slots], g, mask=valid)
            plsc.store_scatter(pose_v, [tok], slots, mask=valid)
            return p + plsc.all_reduce_population_count(valid)[0]

        cnt = lax.fori_loop(0, _N // _L, pass2, jnp.int32(0))

        cnt_v[...] = jnp.full((_L,), cnt, jnp.int32)
        pltpu.sync_copy(cnt_v, counts_h.at[pl.ds(e * _L, _L)])
        pltpu.sync_copy(wgtp_v, wgt_h.at[pl.ds(e * _N, _N)])
        pltpu.sync_copy(pose_v, pose_h.at[pl.ds(e * _N, _N)])

        nblocks = (cnt + _BR - 1) // _BR
        nchunks = nblocks * (_BR // _CH)

        def gchunk(m, _):
            base = m * _CH

            def fill(j2, _2):
                lp = base + j2 * _L + iota
                v = srcp_v[pl.ds(base + j2 * _L, _L)]
                idxc_v[pl.ds(j2 * _L, _L)] = jnp.where(lp < cnt, v, 0)
                return 0

            lax.fori_loop(0, _CH // _L, fill, 0)
            pltpu.async_copy(xf_h.at[idxc_v], rows_v, sem).wait()
            pltpu.sync_copy(rows_v, xg_h.at[pl.ds(e * _N + base, _CH)])
            return 0

        lax.fori_loop(0, nchunks, gchunk, 0)


def _build_sc_compact(interpret=False):
    mesh = plsc.VectorSubcoreMesh(core_axis_name="c", subcore_axis_name="s")
    return functools.partial(
        pl.kernel,
        mesh=mesh,
        interpret=interpret,
        compiler_params=pltpu.CompilerParams(needs_layout_passes=False),
        out_type=(
            jax.ShapeDtypeStruct((_E * _L,), jnp.int32),
            jax.ShapeDtypeStruct((_E * _N,), jnp.float32),
            jax.ShapeDtypeStruct((_E * _N,), jnp.int32),
            jax.ShapeDtypeStruct((_E * _N, _D), jnp.float32),
        ),
        scratch_types=[
            pltpu.VMEM((_N,), jnp.int32),
            pltpu.VMEM((_N,), jnp.int32),
            pltpu.VMEM((_N,), jnp.float32),
            pltpu.VMEM((_N,), jnp.float32),
            pltpu.VMEM((_N,), jnp.int32),
            pltpu.VMEM((_N,), jnp.int32),
            pltpu.VMEM((_N,), jnp.float32),
            pltpu.VMEM((_N,), jnp.int32),
            pltpu.VMEM((_L,), jnp.int32),
            pltpu.VMEM((_CH,), jnp.int32),
            pltpu.VMEM((_CH, _D), jnp.float32),
            pltpu.SemaphoreType.DMA,
        ],
    )(_sc_compact_body)


def _ffn_body(counts_ref, xg_ref, w1_ref, b1_ref, w2_ref, b2_ref, g_ref,
              be_ref, wgt_ref, yg_ref):
    g = pl.program_id(0)
    e = jnp.minimum(g // _NB, _E - 1)
    b = g % _NB
    nb = (counts_ref[e] + _BR - 1) // _BR
    is_z = g == _E * _NB

    @pl.when(is_z)
    def _():
        yg_ref[...] = jnp.zeros((_BR, _D), jnp.float32)

    @pl.when((~is_z) & (b < nb))
    def _():
        xb = xg_ref[...]
        h = jnp.dot(xb, w1_ref[0], preferred_element_type=jnp.float32)
        h = jnp.maximum(h + b1_ref[0, 0][None, :], 0.0)
        o = jnp.dot(h, w2_ref[0], preferred_element_type=jnp.float32)
        o = o + b2_ref[0, 0][None, :]
        hh = xb + o
        mu = jnp.mean(hh, axis=-1, keepdims=True)
        var = jnp.mean((hh - mu) ** 2, axis=-1, keepdims=True)
        y = (hh - mu) * jax.lax.rsqrt(var + 1e-6)
        y = y * g_ref[0, 0][None, :] + be_ref[0, 0][None, :]
        yg_ref[...] = y * wgt_ref[0, 0][:, None]


def _ffn(counts, xg, W1, b1, W2, b2, gamma, beta, wgt):
    def eb(g, counts_ref):
        e = jnp.minimum(g // _NB, _E - 1)
        nb = (counts_ref[e] + _BR - 1) // _BR
        b = jnp.minimum(g % _NB, jnp.maximum(nb - 1, 0))
        return e, b

    def xg_map(g, c):
        e, b = eb(g, c)
        return (jnp.where(g == _E * _NB, _E * _NB, e * _NB + b), 0)

    def w_map(g, c):
        e, _ = eb(g, c)
        return (e, 0, 0)

    def wgt_map(g, c):
        e, b = eb(g, c)
        return (e, 0, b)

    grid_spec = pltpu.PrefetchScalarGridSpec(
        num_scalar_prefetch=1,
        grid=(_E * _NB + 1,),
        in_specs=[
            pl.BlockSpec((_BR, _D), lambda g, c: (
                jnp.minimum(xg_map(g, c)[0], _E * _NB - 1), 0)),
            pl.BlockSpec((1, _D, _F), w_map),
            pl.BlockSpec((1, 1, _F), w_map),
            pl.BlockSpec((1, _F, _D), w_map),
            pl.BlockSpec((1, 1, _D), w_map),
            pl.BlockSpec((1, 1, _D), w_map),
            pl.BlockSpec((1, 1, _D), w_map),
            pl.BlockSpec((1, 1, _BR), wgt_map),
        ],
        out_specs=pl.BlockSpec((_BR, _D), xg_map),
    )
    return pl.pallas_call(
        _ffn_body,
        grid_spec=grid_spec,
        out_shape=jax.ShapeDtypeStruct(((_E * _NB + 1) * _BR, _D), jnp.float32),
    )(counts, xg,
      W1, b1.reshape(_E, 1, _F), W2, b2.reshape(_E, 1, _D),
      gamma.reshape(_E, 1, _D), beta.reshape(_E, 1, _D),
      wgt.reshape(_E, 1, _N))


def _sc_combine_body(yg_h, pose_h, idx0_h, idx1_h, out_h,
                     pose_v, i0c_v, i1c_v, gidx0_v, gidx1_v, rows_v, rows1_v,
                     sem, sem1):
    c = lax.axis_index("c")
    s = lax.axis_index("s")
    wid = s * 2 + c
    base = wid * _CH
    iota = lax.broadcasted_iota(jnp.int32, (_L,), 0)

    pltpu.sync_copy(pose_h, pose_v)
    pltpu.sync_copy(idx0_h.at[pl.ds(base, _CH)], i0c_v)
    pltpu.sync_copy(idx1_h.at[pl.ds(base, _CH)], i1c_v)

    for j2 in range(_CH // _L):
        tok = base + j2 * _L + iota
        i0 = i0c_v[pl.ds(j2 * _L, _L)]
        i1 = i1c_v[pl.ds(j2 * _L, _L)]
        p0 = plsc.load_gather(pose_v, [i0 * _N + tok])
        p1 = plsc.load_gather(pose_v, [i1 * _N + tok])
        gidx0_v[pl.ds(j2 * _L, _L)] = jnp.where(p0 >= 0, i0 * _N + p0, _ZROW)
        gidx1_v[pl.ds(j2 * _L, _L)] = jnp.where(p1 >= 0, i1 * _N + p1, _ZROW)

    d0 = pltpu.async_copy(yg_h.at[gidx0_v], rows_v, sem)
    d1 = pltpu.async_copy(yg_h.at[gidx1_v], rows1_v, sem1)
    d0.wait()
    d1.wait()

    def addj(j, _):
        off = j * _L
        for r in range(_CH):
            rows_v[r, pl.ds(off, _L)] = (rows_v[r, pl.ds(off, _L)]
                                         + rows1_v[r, pl.ds(off, _L)])
        return 0

    lax.fori_loop(0, _D // _L, addj, 0)
    pltpu.sync_copy(rows_v, out_h.at[pl.ds(base, _CH)])


def _build_sc_combine(interpret=False):
    mesh = plsc.VectorSubcoreMesh(core_axis_name="c", subcore_axis_name="s")
    return functools.partial(
        pl.kernel,
        mesh=mesh,
        interpret=interpret,
        compiler_params=pltpu.CompilerParams(needs_layout_passes=False),
        out_type=jax.ShapeDtypeStruct((_N, _D), jnp.float32),
        scratch_types=[
            pltpu.VMEM((_E * _N,), jnp.int32),
            pltpu.VMEM((_CH,), jnp.int32),
            pltpu.VMEM((_CH,), jnp.int32),
            pltpu.VMEM((_CH,), jnp.int32),
            pltpu.VMEM((_CH,), jnp.int32),
            pltpu.VMEM((_CH, _D), jnp.float32),
            pltpu.VMEM((_CH, _D), jnp.float32),
            pltpu.SemaphoreType.DMA,
            pltpu.SemaphoreType.DMA,
        ],
    )(_sc_combine_body)


def kernel(x, Wr, W1, b1, W2, b2, gamma, beta):
    B, S, D = x.shape
    xf = x.reshape(_N, _D)
    idx0, idx1, g0, g1 = _router(xf, Wr)
    counts16, wgt, pose, xg = _build_sc_compact()(idx0, idx1, g0, g1, xf)
    counts = counts16.reshape(_E, _L)[:, 0]
    yg = _ffn(counts, xg, W1, b1, W2, b2, gamma, beta, wgt)
    out = _build_sc_combine()(yg, pose, idx0, idx1)
    return out.reshape(B, S, D)
